# Initial kernel scaffold; baseline (speedup 1.0000x reference)
import jax
import jax.numpy as jnp
from jax.experimental import pallas as pl


def kernel(x, Wr, br, W1, b1, W2, b2):
    raise NotImplementedError("write your pallas kernel here")


# speedup vs baseline: 9.2058x; 9.2058x over previous
import functools

import jax
import jax.numpy as jnp
from jax.experimental import pallas as pl
from jax.experimental.pallas import tpu as pltpu

_M = 128
_TOPK = 2


def _ffn_body(te_ref, na_ref, xs_ref, w1_ref, b1_ref, w2_ref, b2_ref,
              wd_ref, ys_ref):
    i = pl.program_id(0)

    @pl.when(i < na_ref[0])
    def _():
        h = jnp.dot(xs_ref[...], w1_ref[0], preferred_element_type=jnp.float32)
        h = h + b1_ref[0]
        g = 0.5 * h * (1.0 + jax.lax.erf(h * 0.7071067811865476))
        o = jnp.dot(g, w2_ref[0], preferred_element_type=jnp.float32)
        o = o + b2_ref[0]
        ys_ref[...] = o * wd_ref[:, 0:1]


def _grouped_ffn(te, na, xs, W1, b1, W2, b2, wd):
    E, H, F = W1.shape
    R = xs.shape[0]
    NT = R // _M
    grid_spec = pltpu.PrefetchScalarGridSpec(
        num_scalar_prefetch=2,
        grid=(NT,),
        in_specs=[
            pl.BlockSpec((_M, H), lambda i, te, na: (i, 0)),
            pl.BlockSpec((1, H, F), lambda i, te, na: (te[i], 0, 0)),
            pl.BlockSpec((1, 1, F), lambda i, te, na: (te[i], 0, 0)),
            pl.BlockSpec((1, F, H), lambda i, te, na: (te[i], 0, 0)),
            pl.BlockSpec((1, 1, H), lambda i, te, na: (te[i], 0, 0)),
            pl.BlockSpec((_M, 16), lambda i, te, na: (i, 0)),
        ],
        out_specs=pl.BlockSpec((_M, H), lambda i, te, na: (i, 0)),
    )
    return pl.pallas_call(
        _ffn_body,
        grid_spec=grid_spec,
        out_shape=jax.ShapeDtypeStruct((R, H), jnp.float32),
    )(te, na, xs, W1, b1[:, None, :], W2, b2[:, None, :], wd)


def kernel(x, Wr, br, W1, b1, W2, b2):
    b, s, h = x.shape
    e = Wr.shape[1]
    T = b * s
    x_flat = x.reshape(T, h)

    logits = x_flat @ Wr + br
    top_vals, top_idx = jax.lax.top_k(logits, _TOPK)
    w = jax.nn.softmax(top_vals, axis=-1)
    e0 = top_idx[:, 0]
    e1 = top_idx[:, 1]

    iota = jnp.arange(e, dtype=jnp.int32)[None, :]
    oh0 = (e0[:, None] == iota).astype(jnp.float32)
    oh1 = (e1[:, None] == iota).astype(jnp.float32)
    comb = oh0 + oh1
    S = jnp.cumsum(comb, axis=0) - comb
    counts = S[-1] + comb[-1]
    rank0 = jnp.sum(S * oh0, axis=1)
    rank1 = jnp.sum(S * oh1, axis=1)
    padded = jnp.ceil(counts / _M) * _M
    off = jnp.cumsum(padded) - padded
    total = jnp.sum(padded)
    p0 = (jnp.sum(oh0 * off[None, :], axis=1) + rank0).astype(jnp.int32)
    p1 = (jnp.sum(oh1 * off[None, :], axis=1) + rank1).astype(jnp.int32)

    NT = (_TOPK * T) // _M + e
    R = NT * _M
    row = jnp.minimum(jnp.arange(NT, dtype=jnp.float32) * _M, total - _M)
    te = (jnp.sum(off[None, :] <= row[:, None], axis=1) - 1).astype(jnp.int32)
    na = (total / _M).astype(jnp.int32).reshape(1)

    xs = jnp.zeros((R, h), x.dtype).at[p0].set(x_flat).at[p1].set(x_flat)
    wd = (jnp.zeros((R, 16), jnp.float32)
          .at[p0].set(jnp.broadcast_to(w[:, 0:1], (T, 16)))
          .at[p1].set(jnp.broadcast_to(w[:, 1:2], (T, 16))))

    ys = _grouped_ffn(te, na, xs, W1, b1, W2, b2, wd)

    out = (ys[p0] + ys[p1]).reshape(b, s, h)

    usage = counts / jnp.sum(counts)
    loss = jnp.mean((usage - 1.0 / e) ** 2) * 0.01
    return out, loss


# speedup vs baseline: 9.4167x; 1.0229x over previous
import functools

import jax
import jax.numpy as jnp
from jax.experimental import pallas as pl
from jax.experimental.pallas import tpu as pltpu

_M = 128
_TOPK = 2


def _ffn_body(te_ref, na_ref, xs_ref, w1_ref, b1_ref, w2_ref, b2_ref,
              wd_ref, ys_ref):
    i = pl.program_id(0)

    @pl.when(i < na_ref[0])
    def _():
        h = jnp.dot(xs_ref[...], w1_ref[0], preferred_element_type=jnp.float32)
        h = h + b1_ref[0]
        g = 0.5 * h * (1.0 + jax.lax.erf(h * 0.7071067811865476))
        o = jnp.dot(g, w2_ref[0], preferred_element_type=jnp.float32)
        o = o + b2_ref[0]
        ys_ref[...] = o * wd_ref[:, 0:1]


def _grouped_ffn(te, na, xs, W1, b1, W2, b2, wd):
    E, H, F = W1.shape
    R = xs.shape[0]
    NT = R // _M
    grid_spec = pltpu.PrefetchScalarGridSpec(
        num_scalar_prefetch=2,
        grid=(NT,),
        in_specs=[
            pl.BlockSpec((_M, H), lambda i, te, na: (i, 0)),
            pl.BlockSpec((1, H, F), lambda i, te, na: (te[i], 0, 0)),
            pl.BlockSpec((1, 1, F), lambda i, te, na: (te[i], 0, 0)),
            pl.BlockSpec((1, F, H), lambda i, te, na: (te[i], 0, 0)),
            pl.BlockSpec((1, 1, H), lambda i, te, na: (te[i], 0, 0)),
            pl.BlockSpec((_M, 16), lambda i, te, na: (i, 0)),
        ],
        out_specs=pl.BlockSpec((_M, H), lambda i, te, na: (i, 0)),
    )
    return pl.pallas_call(
        _ffn_body,
        grid_spec=grid_spec,
        out_shape=jax.ShapeDtypeStruct((R, H), jnp.float32),
    )(te, na, xs, W1, b1[:, None, :], W2, b2[:, None, :], wd)


def _router_body(x_ref, wr_ref, br_ref, p0_ref, p1_ref, w0_ref, w1_ref,
                 te_ref, na_ref, loss_ref):
    T = x_ref.shape[0]
    E = wr_ref.shape[1]
    NT = te_ref.shape[0]
    logits = jnp.dot(x_ref[...], wr_ref[...],
                     preferred_element_type=jnp.float32) + br_ref[...]
    iota = jax.lax.broadcasted_iota(jnp.int32, (T, E), 1)
    big = jnp.int32(E)
    m0 = jnp.max(logits, axis=1, keepdims=True)
    a0 = jnp.min(jnp.where(logits == m0, iota, big), axis=1, keepdims=True)
    masked = jnp.where(iota == a0, -jnp.inf, logits)
    m1 = jnp.max(masked, axis=1, keepdims=True)
    a1 = jnp.min(jnp.where(masked == m1, iota, big), axis=1, keepdims=True)
    sr = jnp.exp(m1 - m0)
    w0 = 1.0 / (1.0 + sr)
    w1 = sr / (1.0 + sr)
    w0_ref[...] = jnp.broadcast_to(w0, w0_ref.shape)
    w1_ref[...] = jnp.broadcast_to(w1, w1_ref.shape)

    oh0 = (iota == a0).astype(jnp.float32)
    oh1 = (iota == a1).astype(jnp.float32)
    comb = oh0 + oh1
    ir = jax.lax.broadcasted_iota(jnp.int32, (T, T), 0)
    ic = jax.lax.broadcasted_iota(jnp.int32, (T, T), 1)
    lstrict = (ic < ir).astype(jnp.bfloat16)
    S = jnp.dot(lstrict, comb.astype(jnp.bfloat16),
                preferred_element_type=jnp.float32)
    counts = jnp.sum(comb, axis=0, keepdims=True)
    padded = jnp.ceil(counts * (1.0 / _M)) * _M
    ur = jax.lax.broadcasted_iota(jnp.int32, (E, E), 0)
    uc = jax.lax.broadcasted_iota(jnp.int32, (E, E), 1)
    ustrict = (ur < uc).astype(jnp.float32)
    off = jnp.dot(padded, ustrict, preferred_element_type=jnp.float32)
    total = jnp.sum(padded, axis=1, keepdims=True)
    rank0 = jnp.sum(S * oh0, axis=1, keepdims=True)
    rank1 = jnp.sum(S * oh1, axis=1, keepdims=True)
    base0 = jnp.sum(off * oh0, axis=1, keepdims=True)
    base1 = jnp.sum(off * oh1, axis=1, keepdims=True)
    p0_ref[...] = (base0 + rank0).astype(jnp.int32)
    p1_ref[...] = (base1 + rank1).astype(jnp.int32)

    tiota = jax.lax.broadcasted_iota(jnp.int32, (NT, E), 0).astype(jnp.float32)
    rowpos = jnp.minimum(tiota * _M, total - _M)
    offb = jnp.broadcast_to(off, (NT, E))
    te_ref[...] = (jnp.sum((offb <= rowpos).astype(jnp.float32), axis=1,
                           keepdims=True) - 1.0).astype(jnp.int32)
    na_ref[...] = (total * (1.0 / _M)).astype(jnp.int32)

    usage = counts / jnp.sum(counts)
    loss_ref[...] = (jnp.sum((usage - 1.0 / E) ** 2, axis=1, keepdims=True)
                     / E * 0.01)


def _router(x_flat, Wr, br, NT):
    T, H = x_flat.shape
    E = Wr.shape[1]
    outs = (
        jax.ShapeDtypeStruct((T, 1), jnp.int32),
        jax.ShapeDtypeStruct((T, 1), jnp.int32),
        jax.ShapeDtypeStruct((T, 16), jnp.float32),
        jax.ShapeDtypeStruct((T, 16), jnp.float32),
        jax.ShapeDtypeStruct((NT, 1), jnp.int32),
        jax.ShapeDtypeStruct((1, 1), jnp.int32),
        jax.ShapeDtypeStruct((1, 1), jnp.float32),
    )
    return pl.pallas_call(_router_body, out_shape=outs)(
        x_flat, Wr, br.reshape(1, E))


def kernel(x, Wr, br, W1, b1, W2, b2):
    b, s, h = x.shape
    e = Wr.shape[1]
    T = b * s
    x_flat = x.reshape(T, h)

    NT = (_TOPK * T) // _M + e
    R = NT * _M

    p0c, p1c, w0b, w1b, tec, nac, lossc = _router(x_flat, Wr, br, NT)
    p0 = p0c.reshape(T)
    p1 = p1c.reshape(T)
    te = tec.reshape(NT)
    na = nac.reshape(1)

    xs = jnp.zeros((R, h), x.dtype).at[p0].set(x_flat).at[p1].set(x_flat)
    wd = (jnp.zeros((R, 16), jnp.float32)
          .at[p0].set(w0b)
          .at[p1].set(w1b))

    ys = _grouped_ffn(te, na, xs, W1, b1, W2, b2, wd)

    out = (ys[p0] + ys[p1]).reshape(b, s, h)

    return out, lossc.reshape(())


# speedup vs baseline: 10.8493x; 1.1521x over previous
import functools

import jax
import jax.numpy as jnp
from jax import lax
from jax.experimental import pallas as pl
from jax.experimental.pallas import tpu as pltpu
from jax.experimental.pallas import tpu_sc as plsc

_M = 128
_TOPK = 2


def _ffn_body(te_ref, na_ref, xs_ref, w1_ref, b1_ref, w2_ref, b2_ref,
              wd_ref, ys_ref):
    i = pl.program_id(0)

    @pl.when(i < na_ref[0])
    def _():
        h = jnp.dot(xs_ref[...], w1_ref[0], preferred_element_type=jnp.float32)
        h = h + b1_ref[0]
        g = 0.5 * h * (1.0 + jax.lax.erf(h * 0.7071067811865476))
        o = jnp.dot(g, w2_ref[0], preferred_element_type=jnp.float32)
        o = o + b2_ref[0]
        ys_ref[...] = o * wd_ref[:, 0:1]


def _grouped_ffn(te, na, xs, W1, b1, W2, b2, wd):
    E, H, F = W1.shape
    R = xs.shape[0]
    NT = R // _M
    grid_spec = pltpu.PrefetchScalarGridSpec(
        num_scalar_prefetch=2,
        grid=(NT,),
        in_specs=[
            pl.BlockSpec((_M, H), lambda i, te, na: (i, 0)),
            pl.BlockSpec((1, H, F), lambda i, te, na: (te[i], 0, 0)),
            pl.BlockSpec((1, 1, F), lambda i, te, na: (te[i], 0, 0)),
            pl.BlockSpec((1, F, H), lambda i, te, na: (te[i], 0, 0)),
            pl.BlockSpec((1, 1, H), lambda i, te, na: (te[i], 0, 0)),
            pl.BlockSpec((_M, 128), lambda i, te, na: (i, 0)),
        ],
        out_specs=pl.BlockSpec((_M, H), lambda i, te, na: (i, 0)),
    )
    return pl.pallas_call(
        _ffn_body,
        grid_spec=grid_spec,
        out_shape=jax.ShapeDtypeStruct((R, H), jnp.float32),
    )(te, na, xs, W1, b1[:, None, :], W2, b2[:, None, :], wd)


def _router_body(x_ref, wr_ref, br_ref, p0_ref, p1_ref, w0_ref, w1_ref,
                 te_ref, na_ref, loss_ref):
    T = x_ref.shape[0]
    E = wr_ref.shape[1]
    NT = te_ref.shape[0]
    logits = jnp.dot(x_ref[...], wr_ref[...],
                     preferred_element_type=jnp.float32) + br_ref[...]
    iota = jax.lax.broadcasted_iota(jnp.int32, (T, E), 1)
    big = jnp.int32(E)
    m0 = jnp.max(logits, axis=1, keepdims=True)
    a0 = jnp.min(jnp.where(logits == m0, iota, big), axis=1, keepdims=True)
    masked = jnp.where(iota == a0, -jnp.inf, logits)
    m1 = jnp.max(masked, axis=1, keepdims=True)
    a1 = jnp.min(jnp.where(masked == m1, iota, big), axis=1, keepdims=True)
    sr = jnp.exp(m1 - m0)
    w0 = 1.0 / (1.0 + sr)
    w1 = sr / (1.0 + sr)
    w0_ref[...] = jnp.broadcast_to(w0, w0_ref.shape)
    w1_ref[...] = jnp.broadcast_to(w1, w1_ref.shape)

    oh0 = (iota == a0).astype(jnp.float32)
    oh1 = (iota == a1).astype(jnp.float32)
    comb = oh0 + oh1
    ir = jax.lax.broadcasted_iota(jnp.int32, (T, T), 0)
    ic = jax.lax.broadcasted_iota(jnp.int32, (T, T), 1)
    lstrict = (ic < ir).astype(jnp.bfloat16)
    S = jnp.dot(lstrict, comb.astype(jnp.bfloat16),
                preferred_element_type=jnp.float32)
    counts = jnp.sum(comb, axis=0, keepdims=True)
    padded = jnp.ceil(counts * (1.0 / _M)) * _M
    ur = jax.lax.broadcasted_iota(jnp.int32, (E, E), 0)
    uc = jax.lax.broadcasted_iota(jnp.int32, (E, E), 1)
    ustrict = (ur < uc).astype(jnp.float32)
    off = jnp.dot(padded, ustrict, preferred_element_type=jnp.float32)
    total = jnp.sum(padded, axis=1, keepdims=True)
    rank0 = jnp.sum(S * oh0, axis=1, keepdims=True)
    rank1 = jnp.sum(S * oh1, axis=1, keepdims=True)
    base0 = jnp.sum(off * oh0, axis=1, keepdims=True)
    base1 = jnp.sum(off * oh1, axis=1, keepdims=True)
    p0_ref[...] = (base0 + rank0).astype(jnp.int32)
    p1_ref[...] = (base1 + rank1).astype(jnp.int32)

    tiota = jax.lax.broadcasted_iota(jnp.int32, (NT, E), 0).astype(jnp.float32)
    rowpos = jnp.minimum(tiota * _M, total - _M)
    offb = jnp.broadcast_to(off, (NT, E))
    te_ref[...] = (jnp.sum((offb <= rowpos).astype(jnp.float32), axis=1,
                           keepdims=True) - 1.0).astype(jnp.int32)
    na_ref[...] = (total * (1.0 / _M)).astype(jnp.int32)

    usage = counts / jnp.sum(counts)
    loss_ref[...] = (jnp.sum((usage - 1.0 / E) ** 2, axis=1, keepdims=True)
                     / E * 0.01)


def _router(x_flat, Wr, br, NT):
    T, H = x_flat.shape
    E = Wr.shape[1]
    outs = (
        jax.ShapeDtypeStruct((T, 1), jnp.int32),
        jax.ShapeDtypeStruct((T, 1), jnp.int32),
        jax.ShapeDtypeStruct((T, 128), jnp.float32),
        jax.ShapeDtypeStruct((T, 128), jnp.float32),
        jax.ShapeDtypeStruct((NT, 1), jnp.int32),
        jax.ShapeDtypeStruct((1, 1), jnp.int32),
        jax.ShapeDtypeStruct((1, 1), jnp.float32),
    )
    return pl.pallas_call(_router_body, out_shape=outs)(
        x_flat, Wr, br.reshape(1, E))


_NW = 32


def _dispatch_scatter(x_flat, ws, p3, R):
    T, H = x_flat.shape
    CH = (_TOPK * T) // _NW
    wins = T // CH
    mesh = plsc.VectorSubcoreMesh(core_axis_name="c", subcore_axis_name="s")

    @functools.partial(
        pl.kernel,
        mesh=mesh,
        out_type=(jax.ShapeDtypeStruct((R, H), jnp.float32),
                  jax.ShapeDtypeStruct((R, 128), jnp.float32)),
        scratch_types=[
            pltpu.VMEM((1, CH), jnp.int32),
            pltpu.VMEM((CH, H), jnp.float32),
            pltpu.VMEM((CH, 128), jnp.float32),
        ],
    )
    def k(x_hbm, ws_hbm, p_hbm, xs_hbm, wd_hbm, idx_v, xv, wv):
        wid = lax.axis_index("s") * 2 + lax.axis_index("c")
        base = wid * CH
        tokbase = lax.rem(wid, wins) * CH
        pltpu.sync_copy(p_hbm.at[wid], idx_v)
        pltpu.sync_copy(x_hbm.at[pl.ds(tokbase, CH)], xv)
        pltpu.sync_copy(ws_hbm.at[pl.ds(base, CH)], wv)
        pltpu.sync_copy(xv, xs_hbm.at[idx_v.at[0]])
        pltpu.sync_copy(wv, wd_hbm.at[idx_v.at[0]])

    return k(x_flat, ws, p3)


def _combine(ys, p2, T):
    H = ys.shape[1]
    CH = T // _NW
    mesh = plsc.VectorSubcoreMesh(core_axis_name="c", subcore_axis_name="s")

    @functools.partial(
        pl.kernel,
        mesh=mesh,
        out_type=jax.ShapeDtypeStruct((T, H), jnp.float32),
        scratch_types=[
            pltpu.VMEM((1, CH), jnp.int32),
            pltpu.VMEM((1, CH), jnp.int32),
            pltpu.VMEM((CH, H), jnp.float32),
            pltpu.VMEM((CH, H), jnp.float32),
        ],
    )
    def k(ys_hbm, p_hbm, out_hbm, i0, i1, g0, g1):
        wid = lax.axis_index("s") * 2 + lax.axis_index("c")
        tb = wid * CH
        pltpu.sync_copy(p_hbm.at[pl.ds(tb, CH)], i0.at[0])
        pltpu.sync_copy(p_hbm.at[pl.ds(T + tb, CH)], i1.at[0])
        pltpu.sync_copy(ys_hbm.at[i0.at[0]], g0)
        pltpu.sync_copy(ys_hbm.at[i1.at[0]], g1)

        @pl.loop(0, CH)
        def _(r):
            @pl.loop(0, H, step=16)
            def _(c):
                g0[r, pl.ds(c, 16)] = g0[r, pl.ds(c, 16)] + g1[r, pl.ds(c, 16)]

        pltpu.sync_copy(g0, out_hbm.at[pl.ds(tb, CH)])

    return k(ys, p2)


def kernel(x, Wr, br, W1, b1, W2, b2):
    b, s, h = x.shape
    e = Wr.shape[1]
    T = b * s
    x_flat = x.reshape(T, h)

    NT = (_TOPK * T) // _M + e
    R = NT * _M

    p0c, p1c, w0b, w1b, tec, nac, lossc = _router(x_flat, Wr, br, NT)
    p0 = p0c.reshape(T)
    p1 = p1c.reshape(T)
    te = tec.reshape(NT)
    na = nac.reshape(1)

    p2 = jnp.concatenate([p0, p1])
    p3 = p2.reshape(_NW, 1, (_TOPK * T) // _NW)
    ws = jnp.concatenate([w0b, w1b], axis=0)
    xs, wd = _dispatch_scatter(x_flat, ws, p3, R)

    ys = _grouped_ffn(te, na, xs, W1, b1, W2, b2, wd)

    out = _combine(ys, p2, T).reshape(b, s, h)

    return out, lossc.reshape(())


# speedup vs baseline: 11.3986x; 1.0506x over previous
import functools

import jax
import jax.numpy as jnp
from jax import lax
from jax.experimental import pallas as pl
from jax.experimental.pallas import tpu as pltpu
from jax.experimental.pallas import tpu_sc as plsc

_M = 128
_TOPK = 2


def _ffn_body(te_ref, na_ref, xs_ref, w1_ref, b1_ref, w2_ref, b2_ref,
              wd_ref, ys_ref):
    i = pl.program_id(0)

    @pl.when(i < na_ref[0])
    def _():
        h = jnp.dot(xs_ref[...], w1_ref[0], preferred_element_type=jnp.float32)
        h = h + b1_ref[0]
        g = 0.5 * h * (1.0 + jax.lax.erf(h * 0.7071067811865476))
        o = jnp.dot(g, w2_ref[0], preferred_element_type=jnp.float32)
        o = o + b2_ref[0]
        ys_ref[...] = o * wd_ref[:, 0:1]


def _grouped_ffn(te, na, xs, W1, b1, W2, b2, wd):
    E, H, F = W1.shape
    R = xs.shape[0]
    NT = R // _M
    grid_spec = pltpu.PrefetchScalarGridSpec(
        num_scalar_prefetch=2,
        grid=(NT,),
        in_specs=[
            pl.BlockSpec((_M, H), lambda i, te, na: (jnp.minimum(i, na[0] - 1), 0)),
            pl.BlockSpec((1, H, F), lambda i, te, na: (te[i], 0, 0)),
            pl.BlockSpec((1, 1, F), lambda i, te, na: (te[i], 0, 0)),
            pl.BlockSpec((1, F, H), lambda i, te, na: (te[i], 0, 0)),
            pl.BlockSpec((1, 1, H), lambda i, te, na: (te[i], 0, 0)),
            pl.BlockSpec((_M, 128), lambda i, te, na: (jnp.minimum(i, na[0] - 1), 0)),
        ],
        out_specs=pl.BlockSpec((_M, H),
                               lambda i, te, na: (jnp.minimum(i, na[0] - 1), 0)),
    )
    return pl.pallas_call(
        _ffn_body,
        grid_spec=grid_spec,
        out_shape=jax.ShapeDtypeStruct((R, H), jnp.float32),
    )(te, na, xs, W1, b1[:, None, :], W2, b2[:, None, :], wd)


def _router_body(x_ref, wr_ref, br_ref, p0_ref, p1_ref, w0_ref, w1_ref,
                 te_ref, na_ref, loss_ref):
    T = x_ref.shape[0]
    E = wr_ref.shape[1]
    NT = te_ref.shape[0]
    logits = jnp.dot(x_ref[...], wr_ref[...],
                     preferred_element_type=jnp.float32) + br_ref[...]
    iota = jax.lax.broadcasted_iota(jnp.int32, (T, E), 1)
    big = jnp.int32(E)
    m0 = jnp.max(logits, axis=1, keepdims=True)
    a0 = jnp.min(jnp.where(logits == m0, iota, big), axis=1, keepdims=True)
    masked = jnp.where(iota == a0, -jnp.inf, logits)
    m1 = jnp.max(masked, axis=1, keepdims=True)
    a1 = jnp.min(jnp.where(masked == m1, iota, big), axis=1, keepdims=True)
    sr = jnp.exp(m1 - m0)
    w0 = 1.0 / (1.0 + sr)
    w1 = sr / (1.0 + sr)
    w0_ref[...] = jnp.broadcast_to(w0, w0_ref.shape)
    w1_ref[...] = jnp.broadcast_to(w1, w1_ref.shape)

    oh0 = (iota == a0).astype(jnp.float32)
    oh1 = (iota == a1).astype(jnp.float32)
    comb = oh0 + oh1
    ir = jax.lax.broadcasted_iota(jnp.int32, (T, T), 0)
    ic = jax.lax.broadcasted_iota(jnp.int32, (T, T), 1)
    lstrict = (ic < ir).astype(jnp.bfloat16)
    S = jnp.dot(lstrict, comb.astype(jnp.bfloat16),
                preferred_element_type=jnp.float32)
    counts = jnp.sum(comb, axis=0, keepdims=True)
    padded = jnp.ceil(counts * (1.0 / _M)) * _M
    ur = jax.lax.broadcasted_iota(jnp.int32, (E, E), 0)
    uc = jax.lax.broadcasted_iota(jnp.int32, (E, E), 1)
    ustrict = (ur < uc).astype(jnp.float32)
    off = jnp.dot(padded, ustrict, preferred_element_type=jnp.float32)
    total = jnp.sum(padded, axis=1, keepdims=True)
    rank0 = jnp.sum(S * oh0, axis=1, keepdims=True)
    rank1 = jnp.sum(S * oh1, axis=1, keepdims=True)
    base0 = jnp.sum(off * oh0, axis=1, keepdims=True)
    base1 = jnp.sum(off * oh1, axis=1, keepdims=True)
    p0_ref[...] = (base0 + rank0).astype(jnp.int32)
    p1_ref[...] = (base1 + rank1).astype(jnp.int32)

    tiota = jax.lax.broadcasted_iota(jnp.int32, (NT, E), 0).astype(jnp.float32)
    rowpos = jnp.minimum(tiota * _M, total - _M)
    offb = jnp.broadcast_to(off, (NT, E))
    te_ref[...] = (jnp.sum((offb <= rowpos).astype(jnp.float32), axis=1,
                           keepdims=True) - 1.0).astype(jnp.int32)
    na_ref[...] = (total * (1.0 / _M)).astype(jnp.int32)

    usage = counts / jnp.sum(counts)
    loss_ref[...] = (jnp.sum((usage - 1.0 / E) ** 2, axis=1, keepdims=True)
                     / E * 0.01)


def _router(x_flat, Wr, br, NT):
    T, H = x_flat.shape
    E = Wr.shape[1]
    outs = (
        jax.ShapeDtypeStruct((T, 1), jnp.int32),
        jax.ShapeDtypeStruct((T, 1), jnp.int32),
        jax.ShapeDtypeStruct((T, 128), jnp.float32),
        jax.ShapeDtypeStruct((T, 128), jnp.float32),
        jax.ShapeDtypeStruct((NT, 1), jnp.int32),
        jax.ShapeDtypeStruct((1, 1), jnp.int32),
        jax.ShapeDtypeStruct((1, 1), jnp.float32),
    )
    return pl.pallas_call(_router_body, out_shape=outs)(
        x_flat, Wr, br.reshape(1, E))


_NW = 32


def _dispatch_scatter(x_flat, ws, p3, R):
    T, H = x_flat.shape
    CH = (_TOPK * T) // _NW
    wins = T // CH
    mesh = plsc.VectorSubcoreMesh(core_axis_name="c", subcore_axis_name="s")

    @functools.partial(
        pl.kernel,
        mesh=mesh,
        out_type=(jax.ShapeDtypeStruct((R, H), jnp.float32),
                  jax.ShapeDtypeStruct((R, 128), jnp.float32)),
        scratch_types=[
            pltpu.VMEM((1, CH), jnp.int32),
            pltpu.VMEM((CH, H), jnp.float32),
            pltpu.VMEM((CH, 128), jnp.float32),
        ],
    )
    def k(x_hbm, ws_hbm, p_hbm, xs_hbm, wd_hbm, idx_v, xv, wv):
        wid = lax.axis_index("s") * 2 + lax.axis_index("c")
        base = wid * CH
        tokbase = lax.rem(wid, wins) * CH
        pltpu.sync_copy(p_hbm.at[wid], idx_v)
        pltpu.sync_copy(x_hbm.at[pl.ds(tokbase, CH)], xv)
        pltpu.sync_copy(ws_hbm.at[pl.ds(base, CH)], wv)
        pltpu.sync_copy(xv, xs_hbm.at[idx_v.at[0]])
        pltpu.sync_copy(wv, wd_hbm.at[idx_v.at[0]])

    return k(x_flat, ws, p3)


def _combine(ys, p2, T):
    H = ys.shape[1]
    CH = T // _NW
    mesh = plsc.VectorSubcoreMesh(core_axis_name="c", subcore_axis_name="s")

    @functools.partial(
        pl.kernel,
        mesh=mesh,
        out_type=jax.ShapeDtypeStruct((T, H), jnp.float32),
        scratch_types=[
            pltpu.VMEM((1, CH), jnp.int32),
            pltpu.VMEM((1, CH), jnp.int32),
            pltpu.VMEM((CH, H), jnp.float32),
            pltpu.VMEM((CH, H), jnp.float32),
        ],
    )
    def k(ys_hbm, p_hbm, out_hbm, i0, i1, g0, g1):
        wid = lax.axis_index("s") * 2 + lax.axis_index("c")
        tb = wid * CH
        pltpu.sync_copy(p_hbm.at[pl.ds(tb, CH)], i0.at[0])
        pltpu.sync_copy(p_hbm.at[pl.ds(T + tb, CH)], i1.at[0])
        pltpu.sync_copy(ys_hbm.at[i0.at[0]], g0)
        pltpu.sync_copy(ys_hbm.at[i1.at[0]], g1)

        @pl.loop(0, CH)
        def _(r):
            @pl.loop(0, H, step=16)
            def _(c):
                g0[r, pl.ds(c, 16)] = g0[r, pl.ds(c, 16)] + g1[r, pl.ds(c, 16)]

        pltpu.sync_copy(g0, out_hbm.at[pl.ds(tb, CH)])

    return k(ys, p2)


def kernel(x, Wr, br, W1, b1, W2, b2):
    b, s, h = x.shape
    e = Wr.shape[1]
    T = b * s
    x_flat = x.reshape(T, h)

    NT = (_TOPK * T) // _M + e
    R = NT * _M

    p0c, p1c, w0b, w1b, tec, nac, lossc = _router(x_flat, Wr, br, NT)
    p0 = p0c.reshape(T)
    p1 = p1c.reshape(T)
    te = tec.reshape(NT)
    na = nac.reshape(1)

    p2 = jnp.concatenate([p0, p1])
    p3 = p2.reshape(_NW, 1, (_TOPK * T) // _NW)
    ws = jnp.concatenate([w0b, w1b], axis=0)
    xs, wd = _dispatch_scatter(x_flat, ws, p3, R)

    ys = _grouped_ffn(te, na, xs, W1, b1, W2, b2, wd)

    out = _combine(ys, p2, T).reshape(b, s, h)

    return out, lossc.reshape(())
